# baseline (speedup 1.0000x reference)
import functools

import jax
import jax.numpy as jnp
from jax import lax
from jax.experimental import pallas as pl
from jax.experimental.pallas import tpu as pltpu
from jax.experimental.pallas import tpu_sc as plsc

N = 10000
E = 320000
DIN = 128
HID = 256
NCLS = 40
CPAD = 48

NC, NS, L = 2, 16, 16
NW = NC * NS
CH = 128
EPAD = 327680
RPT = EPAD // (NW * CH)
NPAD = 10240
RSUB = NPAD // NS
DEGW = 16

_mesh = plsc.VectorSubcoreMesh(core_axis_name="c", subcore_axis_name="s")
_cp_linear = pltpu.CompilerParams(use_tc_tiling_on_sc=False)


@functools.partial(
    pl.kernel,
    out_type=jax.ShapeDtypeStruct((NC, 2, NPAD, DEGW), jnp.float32),
    mesh=_mesh,
    scratch_types=[
        pltpu.VMEM((RPT, CH), jnp.int32),
        pltpu.VMEM((RPT, CH), jnp.int32),
        pltpu.VMEM((CH, DEGW), jnp.float32),
        pltpu.VMEM((CH, DEGW), jnp.float32),
        pltpu.VMEM_SHARED((NPAD, DEGW), jnp.float32),
        pltpu.VMEM_SHARED((NPAD, DEGW), jnp.float32),
        pltpu.SemaphoreType.DMA,
        pltpu.SemaphoreType.DMA,
    ],
    compiler_params=_cp_linear,
)
def _sc_degrees(src_hbm, dst_hbm, out_hbm, sidx, didx, ones_v, zeros_v,
                acc_o, acc_i, sem_o, sem_i):
    c = lax.axis_index("c")
    s = lax.axis_index("s")
    wid = c * NS + s

    @pl.loop(0, CH)
    def _(i):
        ones_v[i, pl.ds(0, L)] = jnp.ones((L,), jnp.float32)
        zeros_v[i, pl.ds(0, L)] = jnp.zeros((L,), jnp.float32)

    @pl.loop(0, RSUB, step=CH)
    def _(r):
        pltpu.sync_copy(zeros_v, acc_o.at[pl.ds(s * RSUB + r, CH)])
        pltpu.sync_copy(zeros_v, acc_i.at[pl.ds(s * RSUB + r, CH)])

    pltpu.sync_copy(src_hbm.at[pl.ds(wid * RPT, RPT)], sidx)
    pltpu.sync_copy(dst_hbm.at[pl.ds(wid * RPT, RPT)], didx)
    plsc.subcore_barrier()

    @pl.loop(0, RPT)
    def _(j):
        pltpu.async_copy(ones_v, acc_o.at[sidx.at[j]], sem_o, add=True)
        pltpu.async_copy(ones_v, acc_i.at[didx.at[j]], sem_i, add=True)
        pltpu.make_async_copy(ones_v, acc_o.at[sidx.at[j]], sem_o).wait()
        pltpu.make_async_copy(ones_v, acc_i.at[didx.at[j]], sem_i).wait()

    plsc.subcore_barrier()
    pltpu.sync_copy(acc_o.at[pl.ds(s * RSUB, RSUB)],
                    out_hbm.at[c].at[0].at[pl.ds(s * RSUB, RSUB)])
    pltpu.sync_copy(acc_i.at[pl.ds(s * RSUB, RSUB)],
                    out_hbm.at[c].at[1].at[pl.ds(s * RSUB, RSUB)])


def _make_sc_seg_sum(width, ib, split):
    cpt = (2 * RPT) if split else RPT
    assert cpt % ib == 0 and ib % 8 == 0

    @functools.partial(
        pl.kernel,
        out_type=jax.ShapeDtypeStruct((NC, NPAD, width), jnp.float32),
        mesh=_mesh,
        scratch_types=[
            pltpu.VMEM((ib, CH), jnp.int32),
            pltpu.VMEM((ib, CH), jnp.int32),
            pltpu.VMEM((CH, width), jnp.float32),
            pltpu.VMEM((CH, width), jnp.float32),
            pltpu.VMEM((CH, width), jnp.float32),
            pltpu.VMEM((CH, width), jnp.float32),
            pltpu.VMEM_SHARED((NPAD, width), jnp.float32),
            pltpu.VMEM_SHARED((NPAD, width), jnp.float32),
            pltpu.SemaphoreType.DMA,
            pltpu.SemaphoreType.DMA,
            pltpu.SemaphoreType.DMA,
            pltpu.SemaphoreType.DMA,
            pltpu.SemaphoreType.DMA,
            pltpu.SemaphoreType.DMA,
            pltpu.SemaphoreType.DMA,
            pltpu.SemaphoreType.DMA,
        ],
        compiler_params=_cp_linear,
    )
    def seg(taba_hbm, tabb_hbm, src_hbm, dst_hbm, out_hbm, sidx, didx,
            rows0, rows1, rows2, rows3, tab, acc,
            gs0, gs1, gs2, gs3, ss0, ss1, ss2, ss3):
        c = lax.axis_index("c")
        s = lax.axis_index("s")
        off = s * cpt if split else (c * NS + s) * cpt
        bufs = ((rows0, gs0, ss0), (rows1, gs1, ss1),
                (rows2, gs2, ss2), (rows3, gs3, ss3))
        nb = len(bufs)

        @pl.loop(0, CH)
        def _(i):
            @pl.loop(0, width, step=L)
            def _(j):
                rows0[i, pl.ds(j, L)] = jnp.zeros((L,), jnp.float32)

        @pl.loop(0, RSUB, step=CH)
        def _(r):
            pltpu.sync_copy(rows0, acc.at[pl.ds(s * RSUB + r, CH)])

        rsl = pl.ds(s * RSUB, RSUB)

        @pl.when(c == 0)
        def _():
            pltpu.sync_copy(taba_hbm.at[rsl], tab.at[rsl])

        @pl.when(c == 1)
        def _():
            pltpu.sync_copy(tabb_hbm.at[rsl], tab.at[rsl])

        plsc.subcore_barrier()

        @pl.loop(0, cpt // ib)
        def _(g):
            base = off + g * ib
            pltpu.sync_copy(src_hbm.at[pl.ds(base, ib)], sidx)
            pltpu.sync_copy(dst_hbm.at[pl.ds(base, ib)], didx)

            for b, (rb, gs, _) in enumerate(bufs):
                pltpu.async_copy(tab.at[sidx.at[b]], rb, gs)

            @pl.loop(0, ib, step=nb)
            def _(j):
                for b, (rb, gs, ss) in enumerate(bufs):
                    i = j + b
                    pltpu.make_async_copy(tab.at[sidx.at[i]], rb, gs).wait()
                    pltpu.async_copy(rb, acc.at[didx.at[i]], ss, add=True)
                for b, (rb, gs, ss) in enumerate(bufs):
                    i = j + b

                    @pl.when(i + nb < ib)
                    def _():
                        pltpu.make_async_copy(rb, acc.at[didx.at[i]],
                                              ss).wait()
                        pltpu.async_copy(tab.at[sidx.at[i + nb]], rb, gs)

            for b, (rb, _, ss) in enumerate(bufs):
                pltpu.make_async_copy(rb, acc.at[didx.at[ib - nb + b]],
                                      ss).wait()

        plsc.subcore_barrier()
        pltpu.sync_copy(acc.at[rsl], out_hbm.at[c].at[rsl])

    return seg


_sc_seg_sum_h64 = _make_sc_seg_sum(DIN // 2, 32, True)
_sc_seg_sum_48 = _make_sc_seg_sum(CPAD, 16, False)


_RB = 2048
_GRID = NPAD // _RB
_RBF = 1000
_GRIDF = N // _RBF


def _norm(col):
    return lax.rsqrt(jnp.maximum(col, 1.0))


def _prep_body(deg_ref, x_ref, xa_ref, xb_ref, nrm_ref):
    n_out = _norm(deg_ref[0, 0, :, 0:1] + deg_ref[1, 0, :, 0:1])
    n_in = _norm(deg_ref[0, 1, :, 0:1] + deg_ref[1, 1, :, 0:1])
    nrm_ref[:, 0:1] = n_out
    nrm_ref[:, 1:2] = n_in
    xn = x_ref[...] * n_out
    xa_ref[...] = xn[:, : DIN // 2]
    xb_ref[...] = xn[:, DIN // 2 :]


def _tc_prep(degp, x_pad):
    return pl.pallas_call(
        _prep_body,
        grid=(_GRID,),
        in_specs=[
            pl.BlockSpec((NC, 2, _RB, DEGW), lambda i: (0, 0, i, 0)),
            pl.BlockSpec((_RB, DIN), lambda i: (i, 0)),
        ],
        out_specs=[
            pl.BlockSpec((_RB, DIN // 2), lambda i: (i, 0)),
            pl.BlockSpec((_RB, DIN // 2), lambda i: (i, 0)),
            pl.BlockSpec((_RB, 2), lambda i: (i, 0)),
        ],
        out_shape=[
            jax.ShapeDtypeStruct((NPAD, DIN // 2), jnp.float32),
            jax.ShapeDtypeStruct((NPAD, DIN // 2), jnp.float32),
            jax.ShapeDtypeStruct((NPAD, 2), jnp.float32),
        ],
    )(degp, x_pad)


def _mid_body(nrm_ref, p_ref, w1_ref, b1_ref, w2_ref, o_ref):
    n_in = nrm_ref[:, 1:2]
    n_out = nrm_ref[:, 0:1]
    m = jnp.concatenate([p_ref[0], p_ref[1]], axis=1) * n_in
    h = jnp.dot(m, w1_ref[...], preferred_element_type=jnp.float32)
    h = jnp.maximum(h + b1_ref[...], 0.0)
    z = jnp.dot(h, w2_ref[...], preferred_element_type=jnp.float32)
    o_ref[...] = z * n_out


def _tc_mid(norms, p, w1, b1, w2p):
    return pl.pallas_call(
        _mid_body,
        grid=(_GRID,),
        in_specs=[
            pl.BlockSpec((_RB, 2), lambda i: (i, 0)),
            pl.BlockSpec((NC, _RB, DIN // 2), lambda i: (0, i, 0)),
            pl.BlockSpec((DIN, HID), lambda i: (0, 0)),
            pl.BlockSpec((1, HID), lambda i: (0, 0)),
            pl.BlockSpec((HID, CPAD), lambda i: (0, 0)),
        ],
        out_specs=pl.BlockSpec((_RB, CPAD), lambda i: (i, 0)),
        out_shape=jax.ShapeDtypeStruct((NPAD, CPAD), jnp.float32),
    )(norms, p, w1, b1, w2p)


def _final_body(nrm_ref, q_ref, b2_ref, o_ref):
    n_in = nrm_ref[:, 1:2]
    o_ref[...] = ((q_ref[0] + q_ref[1]) * n_in)[:, :NCLS] + b2_ref[...]


def _tc_final(norms, q, b2p):
    return pl.pallas_call(
        _final_body,
        grid=(_GRIDF,),
        in_specs=[
            pl.BlockSpec((_RBF, 2), lambda i: (i, 0)),
            pl.BlockSpec((NC, _RBF, CPAD), lambda i: (0, i, 0)),
            pl.BlockSpec((1, NCLS), lambda i: (0, 0)),
        ],
        out_specs=pl.BlockSpec((_RBF, NCLS), lambda i: (i, 0)),
        out_shape=jax.ShapeDtypeStruct((N, NCLS), jnp.float32),
    )(norms, q, b2p)


def kernel(x, edge_index, W1, b1, W2, b2):
    ei = edge_index.astype(jnp.int32)
    pad = jnp.full((2, EPAD - E), N, dtype=jnp.int32)
    ep = jnp.concatenate([ei, pad], axis=1)
    src_rows = ep[0].reshape(EPAD // CH, CH)
    dst_rows = ep[1].reshape(EPAD // CH, CH)

    degp = _sc_degrees(src_rows, dst_rows)
    x_pad = jnp.pad(x, ((0, NPAD - N), (0, 0)))
    xna, xnb, norms = _tc_prep(degp, x_pad)
    p = _sc_seg_sum_h64(xna, xnb, src_rows, dst_rows)

    w2p = jnp.pad(W2, ((0, 0), (0, CPAD - NCLS)))
    zn = _tc_mid(norms, p, W1, b1.reshape(1, HID), w2p)
    q = _sc_seg_sum_48(zn, zn, src_rows, dst_rows)

    return _tc_final(norms, q, b2.reshape(1, NCLS))


# speedup vs baseline: 1.0550x; 1.0550x over previous
import functools

import jax
import jax.numpy as jnp
from jax import lax
from jax.experimental import pallas as pl
from jax.experimental.pallas import tpu as pltpu
from jax.experimental.pallas import tpu_sc as plsc

N = 10000
E = 320000
DIN = 128
HID = 256
NCLS = 40
CPAD = 48

NC, NS, L = 2, 16, 16
NW = NC * NS
CH = 128
EPAD = 327680
RPT = EPAD // (NW * CH)
NPAD = 10240
RSUB = NPAD // NS
DEGW = 16

_mesh = plsc.VectorSubcoreMesh(core_axis_name="c", subcore_axis_name="s")
_cp_linear = pltpu.CompilerParams(use_tc_tiling_on_sc=False)


@functools.partial(
    pl.kernel,
    out_type=jax.ShapeDtypeStruct((NC, 2, NPAD, DEGW), jnp.float32),
    mesh=_mesh,
    scratch_types=[
        pltpu.VMEM((RPT, CH), jnp.int32),
        pltpu.VMEM((RPT, CH), jnp.int32),
        pltpu.VMEM((CH, DEGW), jnp.float32),
        pltpu.VMEM((CH, DEGW), jnp.float32),
        pltpu.VMEM_SHARED((NPAD, DEGW), jnp.float32),
        pltpu.VMEM_SHARED((NPAD, DEGW), jnp.float32),
        pltpu.SemaphoreType.DMA,
        pltpu.SemaphoreType.DMA,
    ],
    compiler_params=_cp_linear,
)
def _sc_degrees(src_hbm, dst_hbm, out_hbm, sidx, didx, ones_v, zeros_v,
                acc_o, acc_i, sem_o, sem_i):
    c = lax.axis_index("c")
    s = lax.axis_index("s")
    wid = c * NS + s

    @pl.loop(0, CH)
    def _(i):
        ones_v[i, pl.ds(0, L)] = jnp.ones((L,), jnp.float32)
        zeros_v[i, pl.ds(0, L)] = jnp.zeros((L,), jnp.float32)

    @pl.loop(0, RSUB, step=CH)
    def _(r):
        pltpu.sync_copy(zeros_v, acc_o.at[pl.ds(s * RSUB + r, CH)])
        pltpu.sync_copy(zeros_v, acc_i.at[pl.ds(s * RSUB + r, CH)])

    pltpu.sync_copy(src_hbm.at[pl.ds(wid * RPT, RPT)], sidx)
    pltpu.sync_copy(dst_hbm.at[pl.ds(wid * RPT, RPT)], didx)
    plsc.subcore_barrier()

    @pl.loop(0, RPT)
    def _(j):
        pltpu.async_copy(ones_v, acc_o.at[sidx.at[j]], sem_o, add=True)
        pltpu.async_copy(ones_v, acc_i.at[didx.at[j]], sem_i, add=True)
        pltpu.make_async_copy(ones_v, acc_o.at[sidx.at[j]], sem_o).wait()
        pltpu.make_async_copy(ones_v, acc_i.at[didx.at[j]], sem_i).wait()

    plsc.subcore_barrier()
    pltpu.sync_copy(acc_o.at[pl.ds(s * RSUB, RSUB)],
                    out_hbm.at[c].at[0].at[pl.ds(s * RSUB, RSUB)])
    pltpu.sync_copy(acc_i.at[pl.ds(s * RSUB, RSUB)],
                    out_hbm.at[c].at[1].at[pl.ds(s * RSUB, RSUB)])


def _make_sc_seg_sum(width, ib, split):
    cpt = (2 * RPT) if split else RPT
    assert cpt % ib == 0 and ib % 8 == 0

    @functools.partial(
        pl.kernel,
        out_type=jax.ShapeDtypeStruct((NC, NPAD, width), jnp.float32),
        mesh=_mesh,
        scratch_types=[
            pltpu.VMEM((ib, CH), jnp.int32),
            pltpu.VMEM((ib, CH), jnp.int32),
            pltpu.VMEM((CH, width), jnp.float32),
            pltpu.VMEM((CH, width), jnp.float32),
            pltpu.VMEM_SHARED((NPAD, width), jnp.float32),
            pltpu.VMEM_SHARED((NPAD, width), jnp.float32),
            pltpu.SemaphoreType.DMA,
            pltpu.SemaphoreType.DMA,
            pltpu.SemaphoreType.DMA,
            pltpu.SemaphoreType.DMA,
        ],
        compiler_params=_cp_linear,
    )
    def seg(taba_hbm, tabb_hbm, src_hbm, dst_hbm, out_hbm, sidx, didx,
            rows0, rows1, tab, acc, gs0, gs1, ss0, ss1):
        c = lax.axis_index("c")
        s = lax.axis_index("s")
        off = s * cpt if split else (c * NS + s) * cpt
        bufs = ((rows0, gs0, ss0), (rows1, gs1, ss1))
        nb = len(bufs)

        @pl.loop(0, CH)
        def _(i):
            @pl.loop(0, width, step=L)
            def _(j):
                rows0[i, pl.ds(j, L)] = jnp.zeros((L,), jnp.float32)

        @pl.loop(0, RSUB, step=CH)
        def _(r):
            pltpu.sync_copy(rows0, acc.at[pl.ds(s * RSUB + r, CH)])

        rsl = pl.ds(s * RSUB, RSUB)
        if split:
            @pl.when(c == 0)
            def _():
                pltpu.sync_copy(taba_hbm.at[rsl, pl.ds(0, width)],
                                tab.at[rsl])

            @pl.when(c == 1)
            def _():
                pltpu.sync_copy(taba_hbm.at[rsl, pl.ds(width, width)],
                                tab.at[rsl])
        else:
            pltpu.sync_copy(taba_hbm.at[rsl], tab.at[rsl])
        plsc.subcore_barrier()

        @pl.loop(0, cpt // ib)
        def _(g):
            base = off + g * ib
            pltpu.sync_copy(src_hbm.at[pl.ds(base, ib)], sidx)
            pltpu.sync_copy(dst_hbm.at[pl.ds(base, ib)], didx)

            for b, (rb, gs, _) in enumerate(bufs):
                pltpu.async_copy(tab.at[sidx.at[b]], rb, gs)

            @pl.loop(0, ib, step=nb)
            def _(j):
                for b, (rb, gs, ss) in enumerate(bufs):
                    i = j + b
                    pltpu.make_async_copy(tab.at[sidx.at[i]], rb, gs).wait()
                    pltpu.async_copy(rb, acc.at[didx.at[i]], ss, add=True)

                    @pl.when(i + nb < ib)
                    def _():
                        pltpu.make_async_copy(rb, acc.at[didx.at[i]],
                                              ss).wait()
                        pltpu.async_copy(tab.at[sidx.at[i + nb]], rb, gs)

            for b, (rb, _, ss) in enumerate(bufs):
                pltpu.make_async_copy(rb, acc.at[didx.at[ib - nb + b]],
                                      ss).wait()

        plsc.subcore_barrier()
        pltpu.sync_copy(acc.at[rsl], out_hbm.at[c].at[rsl])

    return seg


_sc_seg_sum_h64 = _make_sc_seg_sum(DIN // 2, 16, True)
_sc_seg_sum_48 = _make_sc_seg_sum(CPAD, 16, False)


_RB = 2048
_GRID = NPAD // _RB
_RBF = 1000
_GRIDF = N // _RBF


def _norm(col):
    return lax.rsqrt(jnp.maximum(col, 1.0))


def _prep_body(deg_ref, x_ref, xn_ref, nrm_ref):
    n_out = _norm(deg_ref[0, 0, :, 0:1] + deg_ref[1, 0, :, 0:1])
    n_in = _norm(deg_ref[0, 1, :, 0:1] + deg_ref[1, 1, :, 0:1])
    nrm_ref[:, 0:1] = n_out
    nrm_ref[:, 1:2] = n_in
    xn_ref[...] = x_ref[...] * n_out


def _tc_prep(degp, x_pad):
    return pl.pallas_call(
        _prep_body,
        grid=(_GRID,),
        in_specs=[
            pl.BlockSpec((NC, 2, _RB, DEGW), lambda i: (0, 0, i, 0)),
            pl.BlockSpec((_RB, DIN), lambda i: (i, 0)),
        ],
        out_specs=[
            pl.BlockSpec((_RB, DIN), lambda i: (i, 0)),
            pl.BlockSpec((_RB, 2), lambda i: (i, 0)),
        ],
        out_shape=[
            jax.ShapeDtypeStruct((NPAD, DIN), jnp.float32),
            jax.ShapeDtypeStruct((NPAD, 2), jnp.float32),
        ],
    )(degp, x_pad)


def _mid_body(nrm_ref, p_ref, w1_ref, b1_ref, w2_ref, o_ref):
    n_in = nrm_ref[:, 1:2]
    n_out = nrm_ref[:, 0:1]
    m = jnp.concatenate([p_ref[0], p_ref[1]], axis=1) * n_in
    h = jnp.dot(m, w1_ref[...], preferred_element_type=jnp.float32)
    h = jnp.maximum(h + b1_ref[...], 0.0)
    z = jnp.dot(h, w2_ref[...], preferred_element_type=jnp.float32)
    o_ref[...] = z * n_out


def _tc_mid(norms, p, w1, b1, w2p):
    return pl.pallas_call(
        _mid_body,
        grid=(_GRID,),
        in_specs=[
            pl.BlockSpec((_RB, 2), lambda i: (i, 0)),
            pl.BlockSpec((NC, _RB, DIN // 2), lambda i: (0, i, 0)),
            pl.BlockSpec((DIN, HID), lambda i: (0, 0)),
            pl.BlockSpec((1, HID), lambda i: (0, 0)),
            pl.BlockSpec((HID, CPAD), lambda i: (0, 0)),
        ],
        out_specs=pl.BlockSpec((_RB, CPAD), lambda i: (i, 0)),
        out_shape=jax.ShapeDtypeStruct((NPAD, CPAD), jnp.float32),
    )(norms, p, w1, b1, w2p)


def _final_body(nrm_ref, q_ref, b2_ref, o_ref):
    n_in = nrm_ref[:, 1:2]
    o_ref[...] = ((q_ref[0] + q_ref[1]) * n_in)[:, :NCLS] + b2_ref[...]


def _tc_final(norms, q, b2p):
    return pl.pallas_call(
        _final_body,
        grid=(_GRIDF,),
        in_specs=[
            pl.BlockSpec((_RBF, 2), lambda i: (i, 0)),
            pl.BlockSpec((NC, _RBF, CPAD), lambda i: (0, i, 0)),
            pl.BlockSpec((1, NCLS), lambda i: (0, 0)),
        ],
        out_specs=pl.BlockSpec((_RBF, NCLS), lambda i: (i, 0)),
        out_shape=jax.ShapeDtypeStruct((N, NCLS), jnp.float32),
    )(norms, q, b2p)


def kernel(x, edge_index, W1, b1, W2, b2):
    ei = edge_index.astype(jnp.int32)
    pad = jnp.full((2, EPAD - E), N, dtype=jnp.int32)
    ep = jnp.concatenate([ei, pad], axis=1)
    src_rows = ep[0].reshape(EPAD // CH, CH)
    dst_rows = ep[1].reshape(EPAD // CH, CH)

    degp = _sc_degrees(src_rows, dst_rows)
    x_pad = jnp.pad(x, ((0, NPAD - N), (0, 0)))
    xn, norms = _tc_prep(degp, x_pad)
    p = _sc_seg_sum_h64(xn, xn, src_rows, dst_rows)

    w2p = jnp.pad(W2, ((0, 0), (0, CPAD - NCLS)))
    zn = _tc_mid(norms, p, W1, b1.reshape(1, HID), w2p)
    q = _sc_seg_sum_48(zn, zn, src_rows, dst_rows)

    return _tc_final(norms, q, b2.reshape(1, NCLS))


# speedup vs baseline: 1.0688x; 1.0131x over previous
import functools

import jax
import jax.numpy as jnp
from jax import lax
from jax.experimental import pallas as pl
from jax.experimental.pallas import tpu as pltpu
from jax.experimental.pallas import tpu_sc as plsc

N = 10000
E = 320000
DIN = 128
HID = 256
NCLS = 40
CPAD = 48

NC, NS, L = 2, 16, 16
NW = NC * NS
CH = 128
EPAD = 327680
RPT = EPAD // (NW * CH)
NPAD = 10240
RSUB = NPAD // NS
DEGW = 16

_mesh = plsc.VectorSubcoreMesh(core_axis_name="c", subcore_axis_name="s")
_cp_linear = pltpu.CompilerParams(use_tc_tiling_on_sc=False)


@functools.partial(
    pl.kernel,
    out_type=jax.ShapeDtypeStruct((NC, 2, NPAD, DEGW), jnp.float32),
    mesh=_mesh,
    scratch_types=[
        pltpu.VMEM((RPT, CH), jnp.int32),
        pltpu.VMEM((RPT, CH), jnp.int32),
        pltpu.VMEM((CH, DEGW), jnp.float32),
        pltpu.VMEM((CH, DEGW), jnp.float32),
        pltpu.VMEM_SHARED((NPAD, DEGW), jnp.float32),
        pltpu.VMEM_SHARED((NPAD, DEGW), jnp.float32),
        pltpu.SemaphoreType.DMA,
        pltpu.SemaphoreType.DMA,
    ],
    compiler_params=_cp_linear,
)
def _sc_degrees(src_hbm, dst_hbm, out_hbm, sidx, didx, ones_v, zeros_v,
                acc_o, acc_i, sem_o, sem_i):
    c = lax.axis_index("c")
    s = lax.axis_index("s")
    wid = c * NS + s

    @pl.loop(0, CH)
    def _(i):
        ones_v[i, pl.ds(0, L)] = jnp.ones((L,), jnp.float32)
        zeros_v[i, pl.ds(0, L)] = jnp.zeros((L,), jnp.float32)

    @pl.loop(0, RSUB, step=CH)
    def _(r):
        pltpu.sync_copy(zeros_v, acc_o.at[pl.ds(s * RSUB + r, CH)])
        pltpu.sync_copy(zeros_v, acc_i.at[pl.ds(s * RSUB + r, CH)])

    pltpu.sync_copy(src_hbm.at[pl.ds(wid * RPT, RPT)], sidx)
    pltpu.sync_copy(dst_hbm.at[pl.ds(wid * RPT, RPT)], didx)
    plsc.subcore_barrier()

    @pl.loop(0, RPT)
    def _(j):
        pltpu.async_copy(ones_v, acc_o.at[sidx.at[j]], sem_o, add=True)
        pltpu.async_copy(ones_v, acc_i.at[didx.at[j]], sem_i, add=True)
        pltpu.make_async_copy(ones_v, acc_o.at[sidx.at[j]], sem_o).wait()
        pltpu.make_async_copy(ones_v, acc_i.at[didx.at[j]], sem_i).wait()

    plsc.subcore_barrier()
    pltpu.sync_copy(acc_o.at[pl.ds(s * RSUB, RSUB)],
                    out_hbm.at[c].at[0].at[pl.ds(s * RSUB, RSUB)])
    pltpu.sync_copy(acc_i.at[pl.ds(s * RSUB, RSUB)],
                    out_hbm.at[c].at[1].at[pl.ds(s * RSUB, RSUB)])


def _make_sc_seg_sum(width, ib, split):
    cpt = (2 * RPT) if split else RPT
    assert cpt % ib == 0 and ib % 8 == 0

    @functools.partial(
        pl.kernel,
        out_type=jax.ShapeDtypeStruct((NC, NPAD, width), jnp.float32),
        mesh=_mesh,
        scratch_types=[
            pltpu.VMEM((ib, CH), jnp.int32),
            pltpu.VMEM((ib, CH), jnp.int32),
            pltpu.VMEM((CH, width), jnp.float32),
            pltpu.VMEM((CH, width), jnp.float32),
            pltpu.VMEM_SHARED((NPAD, width), jnp.float32),
            pltpu.VMEM_SHARED((NPAD, width), jnp.float32),
            pltpu.SemaphoreType.DMA,
            pltpu.SemaphoreType.DMA,
            pltpu.SemaphoreType.DMA,
            pltpu.SemaphoreType.DMA,
        ],
        compiler_params=_cp_linear,
    )
    def seg(taba_hbm, tabb_hbm, src_hbm, dst_hbm, out_hbm, sidx, didx,
            rows0, rows1, tab, acc, gs0, gs1, ss0, ss1):
        c = lax.axis_index("c")
        s = lax.axis_index("s")
        off = s * cpt if split else (c * NS + s) * cpt
        bufs = ((rows0, gs0, ss0), (rows1, gs1, ss1))
        nb = len(bufs)

        @pl.loop(0, CH)
        def _(i):
            @pl.loop(0, width, step=L)
            def _(j):
                rows0[i, pl.ds(j, L)] = jnp.zeros((L,), jnp.float32)

        @pl.loop(0, RSUB, step=CH)
        def _(r):
            pltpu.sync_copy(rows0, acc.at[pl.ds(s * RSUB + r, CH)])

        rsl = pl.ds(s * RSUB, RSUB)
        if split:
            @pl.when(c == 0)
            def _():
                pltpu.sync_copy(taba_hbm.at[rsl, pl.ds(0, width)],
                                tab.at[rsl])

            @pl.when(c == 1)
            def _():
                pltpu.sync_copy(taba_hbm.at[rsl, pl.ds(width, width)],
                                tab.at[rsl])
        else:
            pltpu.sync_copy(taba_hbm.at[rsl, pl.ds(0, width)], tab.at[rsl])
        plsc.subcore_barrier()

        @pl.loop(0, cpt // ib)
        def _(g):
            base = off + g * ib
            pltpu.sync_copy(src_hbm.at[pl.ds(base, ib)], sidx)
            pltpu.sync_copy(dst_hbm.at[pl.ds(base, ib)], didx)

            for b, (rb, gs, _) in enumerate(bufs):
                pltpu.async_copy(tab.at[sidx.at[b]], rb, gs)

            @pl.loop(0, ib, step=nb)
            def _(j):
                for b, (rb, gs, ss) in enumerate(bufs):
                    i = j + b
                    pltpu.make_async_copy(tab.at[sidx.at[i]], rb, gs).wait()
                    pltpu.async_copy(rb, acc.at[didx.at[i]], ss, add=True)

                    @pl.when(i + nb < ib)
                    def _():
                        pltpu.make_async_copy(rb, acc.at[didx.at[i]],
                                              ss).wait()
                        pltpu.async_copy(tab.at[sidx.at[i + nb]], rb, gs)

            for b, (rb, _, ss) in enumerate(bufs):
                pltpu.make_async_copy(rb, acc.at[didx.at[ib - nb + b]],
                                      ss).wait()

        plsc.subcore_barrier()
        pltpu.sync_copy(acc.at[rsl], out_hbm.at[c].at[rsl])

    return seg


_sc_seg_sum_h64 = _make_sc_seg_sum(DIN // 2, 16, True)
_sc_seg_sum_48 = _make_sc_seg_sum(CPAD, 16, False)


_RB = 2048
_GRID = NPAD // _RB
_RBF = 1000
_GRIDF = N // _RBF


def _norm(col):
    return lax.rsqrt(jnp.maximum(col, 1.0))


def _prep_body(deg_ref, x_ref, xn_ref, nrm_ref):
    n_out = _norm(deg_ref[0, 0, :, 0:1] + deg_ref[1, 0, :, 0:1])
    n_in = _norm(deg_ref[0, 1, :, 0:1] + deg_ref[1, 1, :, 0:1])
    nrm_ref[:, 0:1] = n_out
    nrm_ref[:, 1:2] = n_in
    xn_ref[...] = x_ref[...] * n_out


def _tc_prep(degp, x_pad):
    return pl.pallas_call(
        _prep_body,
        grid=(_GRID,),
        in_specs=[
            pl.BlockSpec((NC, 2, _RB, DEGW), lambda i: (0, 0, i, 0)),
            pl.BlockSpec((_RB, DIN), lambda i: (i, 0)),
        ],
        out_specs=[
            pl.BlockSpec((_RB, DIN), lambda i: (i, 0)),
            pl.BlockSpec((_RB, 2), lambda i: (i, 0)),
        ],
        out_shape=[
            jax.ShapeDtypeStruct((NPAD, DIN), jnp.float32),
            jax.ShapeDtypeStruct((NPAD, 2), jnp.float32),
        ],
    )(degp, x_pad)


def _mid_body(nrm_ref, p_ref, w1_ref, b1_ref, w2_ref, o_ref):
    n_in = nrm_ref[:, 1:2]
    n_out = nrm_ref[:, 0:1]
    m = jnp.concatenate([p_ref[0], p_ref[1]], axis=1) * n_in
    h = jnp.dot(m, w1_ref[...], preferred_element_type=jnp.float32)
    h = jnp.maximum(h + b1_ref[...], 0.0)
    z = jnp.dot(h, w2_ref[...], preferred_element_type=jnp.float32)
    o_ref[...] = z * n_out


def _tc_mid(norms, p, w1, b1, w2p):
    return pl.pallas_call(
        _mid_body,
        grid=(_GRID,),
        in_specs=[
            pl.BlockSpec((_RB, 2), lambda i: (i, 0)),
            pl.BlockSpec((NC, _RB, DIN // 2), lambda i: (0, i, 0)),
            pl.BlockSpec((DIN, HID), lambda i: (0, 0)),
            pl.BlockSpec((1, HID), lambda i: (0, 0)),
            pl.BlockSpec((HID, DIN), lambda i: (0, 0)),
        ],
        out_specs=pl.BlockSpec((_RB, DIN), lambda i: (i, 0)),
        out_shape=jax.ShapeDtypeStruct((NPAD, DIN), jnp.float32),
    )(norms, p, w1, b1, w2p)


def _final_body(nrm_ref, q_ref, b2_ref, o_ref):
    n_in = nrm_ref[:, 1:2]
    o_ref[...] = ((q_ref[0] + q_ref[1]) * n_in)[:, :NCLS] + b2_ref[...]


def _tc_final(norms, q, b2p):
    return pl.pallas_call(
        _final_body,
        grid=(_GRIDF,),
        in_specs=[
            pl.BlockSpec((_RBF, 2), lambda i: (i, 0)),
            pl.BlockSpec((NC, _RBF, CPAD), lambda i: (0, i, 0)),
            pl.BlockSpec((1, NCLS), lambda i: (0, 0)),
        ],
        out_specs=pl.BlockSpec((_RBF, NCLS), lambda i: (i, 0)),
        out_shape=jax.ShapeDtypeStruct((N, NCLS), jnp.float32),
    )(norms, q, b2p)


def kernel(x, edge_index, W1, b1, W2, b2):
    ei = edge_index.astype(jnp.int32)
    pad = jnp.full((2, EPAD - E), N, dtype=jnp.int32)
    ep = jnp.concatenate([ei, pad], axis=1)
    src_rows = ep[0].reshape(EPAD // CH, CH)
    dst_rows = ep[1].reshape(EPAD // CH, CH)

    degp = _sc_degrees(src_rows, dst_rows)
    x_pad = jnp.pad(x, ((0, NPAD - N), (0, 0)))
    xn, norms = _tc_prep(degp, x_pad)
    p = _sc_seg_sum_h64(xn, xn, src_rows, dst_rows)

    w2p = jnp.pad(W2, ((0, 0), (0, DIN - NCLS)))
    zn = _tc_mid(norms, p, W1, b1.reshape(1, HID), w2p)
    q = _sc_seg_sum_48(zn, zn, src_rows, dst_rows)

    return _tc_final(norms, q, b2.reshape(1, NCLS))


# speedup vs baseline: 1.1005x; 1.0296x over previous
import functools

import jax
import jax.numpy as jnp
from jax import lax
from jax.experimental import pallas as pl
from jax.experimental.pallas import tpu as pltpu
from jax.experimental.pallas import tpu_sc as plsc

N = 10000
E = 320000
DIN = 128
HID = 256
NCLS = 40
CPAD = 48

NC, NS, L = 2, 16, 16
NW = NC * NS
CH = 128
EPAD = 327680
RPT = EPAD // (NW * CH)
NPAD = 10240
RSUB = NPAD // NS
DEGW = 16

_mesh = plsc.VectorSubcoreMesh(core_axis_name="c", subcore_axis_name="s")
_cp_linear = pltpu.CompilerParams(use_tc_tiling_on_sc=False)


@functools.partial(
    pl.kernel,
    out_type=jax.ShapeDtypeStruct((NC, 2, NPAD, 128), jnp.float32),
    mesh=_mesh,
    scratch_types=[
        pltpu.VMEM((RPT, CH), jnp.int32),
        pltpu.VMEM((RPT, CH), jnp.int32),
        pltpu.VMEM((CH, DEGW), jnp.float32),
        pltpu.VMEM((CH, DEGW), jnp.float32),
        pltpu.VMEM_SHARED((NPAD, DEGW), jnp.float32),
        pltpu.VMEM_SHARED((NPAD, DEGW), jnp.float32),
        pltpu.SemaphoreType.DMA,
        pltpu.SemaphoreType.DMA,
    ],
    compiler_params=_cp_linear,
)
def _sc_degrees(src_hbm, dst_hbm, out_hbm, sidx, didx, ones_v, zeros_v,
                acc_o, acc_i, sem_o, sem_i):
    c = lax.axis_index("c")
    s = lax.axis_index("s")
    wid = c * NS + s

    @pl.loop(0, CH)
    def _(i):
        ones_v[i, pl.ds(0, L)] = jnp.ones((L,), jnp.float32)
        zeros_v[i, pl.ds(0, L)] = jnp.zeros((L,), jnp.float32)

    @pl.loop(0, RSUB, step=CH)
    def _(r):
        pltpu.sync_copy(zeros_v, acc_o.at[pl.ds(s * RSUB + r, CH)])
        pltpu.sync_copy(zeros_v, acc_i.at[pl.ds(s * RSUB + r, CH)])

    pltpu.sync_copy(src_hbm.at[pl.ds(wid * RPT, RPT)], sidx)
    pltpu.sync_copy(dst_hbm.at[pl.ds(wid * RPT, RPT)], didx)
    plsc.subcore_barrier()

    @pl.loop(0, RPT)
    def _(j):
        pltpu.async_copy(ones_v, acc_o.at[sidx.at[j]], sem_o, add=True)
        pltpu.async_copy(ones_v, acc_i.at[didx.at[j]], sem_i, add=True)
        pltpu.make_async_copy(ones_v, acc_o.at[sidx.at[j]], sem_o).wait()
        pltpu.make_async_copy(ones_v, acc_i.at[didx.at[j]], sem_i).wait()

    plsc.subcore_barrier()
    rsl = pl.ds(s * RSUB, RSUB)
    pltpu.sync_copy(acc_o.at[rsl],
                    out_hbm.at[c].at[0].at[rsl, pl.ds(0, DEGW)])
    pltpu.sync_copy(acc_i.at[rsl],
                    out_hbm.at[c].at[1].at[rsl, pl.ds(0, DEGW)])


def _make_sc_seg_sum(width, ib, split):
    cpt = (2 * RPT) if split else RPT
    assert cpt % ib == 0 and ib % 8 == 0

    @functools.partial(
        pl.kernel,
        out_type=jax.ShapeDtypeStruct((NC, NPAD, width), jnp.float32),
        mesh=_mesh,
        scratch_types=[
            pltpu.VMEM((ib, CH), jnp.int32),
            pltpu.VMEM((ib, CH), jnp.int32),
            pltpu.VMEM((CH, width), jnp.float32),
            pltpu.VMEM((CH, width), jnp.float32),
            pltpu.VMEM_SHARED((NPAD, width), jnp.float32),
            pltpu.VMEM_SHARED((NPAD, width), jnp.float32),
            pltpu.SemaphoreType.DMA,
            pltpu.SemaphoreType.DMA,
            pltpu.SemaphoreType.DMA,
            pltpu.SemaphoreType.DMA,
        ],
        compiler_params=_cp_linear,
    )
    def seg(taba_hbm, tabb_hbm, src_hbm, dst_hbm, out_hbm, sidx, didx,
            rows0, rows1, tab, acc, gs0, gs1, ss0, ss1):
        c = lax.axis_index("c")
        s = lax.axis_index("s")
        off = s * cpt if split else (c * NS + s) * cpt
        bufs = ((rows0, gs0, ss0), (rows1, gs1, ss1))
        nb = len(bufs)

        @pl.loop(0, CH)
        def _(i):
            @pl.loop(0, width, step=L)
            def _(j):
                rows0[i, pl.ds(j, L)] = jnp.zeros((L,), jnp.float32)

        @pl.loop(0, RSUB, step=CH)
        def _(r):
            pltpu.sync_copy(rows0, acc.at[pl.ds(s * RSUB + r, CH)])

        rsl = pl.ds(s * RSUB, RSUB)
        if split:
            @pl.when(c == 0)
            def _():
                pltpu.sync_copy(taba_hbm.at[rsl, pl.ds(0, width)],
                                tab.at[rsl])

            @pl.when(c == 1)
            def _():
                pltpu.sync_copy(taba_hbm.at[rsl, pl.ds(width, width)],
                                tab.at[rsl])
        else:
            pltpu.sync_copy(taba_hbm.at[rsl, pl.ds(0, width)], tab.at[rsl])
        plsc.subcore_barrier()

        @pl.loop(0, cpt // ib)
        def _(g):
            base = off + g * ib
            pltpu.sync_copy(src_hbm.at[pl.ds(base, ib)], sidx)
            pltpu.sync_copy(dst_hbm.at[pl.ds(base, ib)], didx)

            for b, (rb, gs, _) in enumerate(bufs):
                pltpu.async_copy(tab.at[sidx.at[b]], rb, gs)

            @pl.loop(0, ib, step=nb)
            def _(j):
                for b, (rb, gs, ss) in enumerate(bufs):
                    i = j + b
                    pltpu.make_async_copy(tab.at[sidx.at[i]], rb, gs).wait()
                    pltpu.async_copy(rb, acc.at[didx.at[i]], ss, add=True)

                    @pl.when(i + nb < ib)
                    def _():
                        pltpu.make_async_copy(rb, acc.at[didx.at[i]],
                                              ss).wait()
                        pltpu.async_copy(tab.at[sidx.at[i + nb]], rb, gs)

            for b, (rb, _, ss) in enumerate(bufs):
                pltpu.make_async_copy(rb, acc.at[didx.at[ib - nb + b]],
                                      ss).wait()

        plsc.subcore_barrier()
        pltpu.sync_copy(acc.at[rsl], out_hbm.at[c].at[rsl])

    return seg


_sc_seg_sum_h64 = _make_sc_seg_sum(DIN // 2, 16, True)
_sc_seg_sum_48 = _make_sc_seg_sum(CPAD, 16, False)


_RB = 2048
_GRID = NPAD // _RB
_RBF = 1000
_GRIDF = N // _RBF


def _norm(col):
    return lax.rsqrt(jnp.maximum(col, 1.0))


def _prep_body(deg_ref, x_ref, xn_ref, nrm_ref):
    n_out = _norm(deg_ref[0, 0, :, 0:1] + deg_ref[1, 0, :, 0:1])
    n_in = _norm(deg_ref[0, 1, :, 0:1] + deg_ref[1, 1, :, 0:1])
    nrm_ref[:, 0:1] = n_out
    nrm_ref[:, 1:2] = n_in
    xn_ref[...] = x_ref[...] * n_out


def _tc_prep(degp, x_pad):
    return pl.pallas_call(
        _prep_body,
        grid=(_GRID,),
        in_specs=[
            pl.BlockSpec((NC, 2, _RB, 128), lambda i: (0, 0, i, 0)),
            pl.BlockSpec((_RB, DIN), lambda i: (i, 0)),
        ],
        out_specs=[
            pl.BlockSpec((_RB, DIN), lambda i: (i, 0)),
            pl.BlockSpec((_RB, 2), lambda i: (i, 0)),
        ],
        out_shape=[
            jax.ShapeDtypeStruct((NPAD, DIN), jnp.float32),
            jax.ShapeDtypeStruct((NPAD, 2), jnp.float32),
        ],
    )(degp, x_pad)


def _mid_body(nrm_ref, p_ref, w1_ref, b1_ref, w2_ref, o_ref):
    n_in = nrm_ref[:, 1:2]
    n_out = nrm_ref[:, 0:1]
    m = jnp.concatenate([p_ref[0], p_ref[1]], axis=1) * n_in
    h = jnp.dot(m, w1_ref[...], preferred_element_type=jnp.float32)
    h = jnp.maximum(h + b1_ref[...], 0.0)
    z = jnp.dot(h, w2_ref[...], preferred_element_type=jnp.float32)
    o_ref[...] = z * n_out


def _tc_mid(norms, p, w1, b1, w2p):
    return pl.pallas_call(
        _mid_body,
        grid=(_GRID,),
        in_specs=[
            pl.BlockSpec((_RB, 2), lambda i: (i, 0)),
            pl.BlockSpec((NC, _RB, DIN // 2), lambda i: (0, i, 0)),
            pl.BlockSpec((DIN, HID), lambda i: (0, 0)),
            pl.BlockSpec((1, HID), lambda i: (0, 0)),
            pl.BlockSpec((HID, DIN), lambda i: (0, 0)),
        ],
        out_specs=pl.BlockSpec((_RB, DIN), lambda i: (i, 0)),
        out_shape=jax.ShapeDtypeStruct((NPAD, DIN), jnp.float32),
    )(norms, p, w1, b1, w2p)


def _final_body(nrm_ref, q_ref, b2_ref, o_ref):
    n_in = nrm_ref[:, 1:2]
    o_ref[...] = ((q_ref[0] + q_ref[1]) * n_in)[:, :NCLS] + b2_ref[...]


def _tc_final(norms, q, b2p):
    return pl.pallas_call(
        _final_body,
        grid=(_GRIDF,),
        in_specs=[
            pl.BlockSpec((_RBF, 2), lambda i: (i, 0)),
            pl.BlockSpec((NC, _RBF, CPAD), lambda i: (0, i, 0)),
            pl.BlockSpec((1, NCLS), lambda i: (0, 0)),
        ],
        out_specs=pl.BlockSpec((_RBF, NCLS), lambda i: (i, 0)),
        out_shape=jax.ShapeDtypeStruct((N, NCLS), jnp.float32),
    )(norms, q, b2p)


def kernel(x, edge_index, W1, b1, W2, b2):
    ei = edge_index.astype(jnp.int32)
    pad = jnp.full((2, EPAD - E), N, dtype=jnp.int32)
    ep = jnp.concatenate([ei, pad], axis=1)
    src_rows = ep[0].reshape(EPAD // CH, CH)
    dst_rows = ep[1].reshape(EPAD // CH, CH)

    degp = _sc_degrees(src_rows, dst_rows)
    x_pad = jnp.pad(x, ((0, NPAD - N), (0, 0)))
    xn, norms = _tc_prep(degp, x_pad)
    p = _sc_seg_sum_h64(xn, xn, src_rows, dst_rows)

    w2p = jnp.pad(W2, ((0, 0), (0, DIN - NCLS)))
    zn = _tc_mid(norms, p, W1, b1.reshape(1, HID), w2p)
    q = _sc_seg_sum_48(zn, zn, src_rows, dst_rows)

    return _tc_final(norms, q, b2.reshape(1, NCLS))


# speedup vs baseline: 1.1591x; 1.0533x over previous
import functools

import jax
import jax.numpy as jnp
from jax import lax
from jax.experimental import pallas as pl
from jax.experimental.pallas import tpu as pltpu
from jax.experimental.pallas import tpu_sc as plsc

N = 10000
E = 320000
DIN = 128
HID = 256
NCLS = 40
CPAD = 48

NC, NS, L = 2, 16, 16
NW = NC * NS
CH = 128
EPAD = 327680
RPT = EPAD // (NW * CH)
NPAD = 10240
RSUB = NPAD // NS
DEGW = 16

_mesh = plsc.VectorSubcoreMesh(core_axis_name="c", subcore_axis_name="s")
_cp_linear = pltpu.CompilerParams(use_tc_tiling_on_sc=False)


@functools.partial(
    pl.kernel,
    out_type=jax.ShapeDtypeStruct((NC, 2, NPAD, 128), jnp.float32),
    mesh=_mesh,
    scratch_types=[
        pltpu.VMEM((RPT, CH), jnp.int32),
        pltpu.VMEM((RPT, CH), jnp.int32),
        pltpu.VMEM((CH, DEGW), jnp.float32),
        pltpu.VMEM((CH, DEGW), jnp.float32),
        pltpu.VMEM_SHARED((NPAD, DEGW), jnp.float32),
        pltpu.VMEM_SHARED((NPAD, DEGW), jnp.float32),
        pltpu.SemaphoreType.DMA,
        pltpu.SemaphoreType.DMA,
    ],
    compiler_params=_cp_linear,
)
def _sc_degrees(src_hbm, dst_hbm, out_hbm, sidx, didx, ones_v, zeros_v,
                acc_o, acc_i, sem_o, sem_i):
    c = lax.axis_index("c")
    s = lax.axis_index("s")
    wid = c * NS + s

    @pl.loop(0, CH)
    def _(i):
        ones_v[i, pl.ds(0, L)] = jnp.ones((L,), jnp.float32)
        zeros_v[i, pl.ds(0, L)] = jnp.zeros((L,), jnp.float32)

    @pl.loop(0, RSUB, step=CH)
    def _(r):
        pltpu.sync_copy(zeros_v, acc_o.at[pl.ds(s * RSUB + r, CH)])
        pltpu.sync_copy(zeros_v, acc_i.at[pl.ds(s * RSUB + r, CH)])

    pltpu.sync_copy(src_hbm.at[pl.ds(wid * RPT, RPT)], sidx)
    pltpu.sync_copy(dst_hbm.at[pl.ds(wid * RPT, RPT)], didx)
    plsc.subcore_barrier()

    @pl.loop(0, RPT)
    def _(j):
        pltpu.async_copy(ones_v, acc_o.at[sidx.at[j]], sem_o, add=True)
        pltpu.async_copy(ones_v, acc_i.at[didx.at[j]], sem_i, add=True)
        pltpu.make_async_copy(ones_v, acc_o.at[sidx.at[j]], sem_o).wait()
        pltpu.make_async_copy(ones_v, acc_i.at[didx.at[j]], sem_i).wait()

    plsc.subcore_barrier()
    rsl = pl.ds(s * RSUB, RSUB)
    pltpu.sync_copy(acc_o.at[rsl],
                    out_hbm.at[c].at[0].at[rsl, pl.ds(0, DEGW)])
    pltpu.sync_copy(acc_i.at[rsl],
                    out_hbm.at[c].at[1].at[rsl, pl.ds(0, DEGW)])


def _make_sc_seg_sum(width, ib, split):
    cpt = (2 * RPT) if split else RPT
    assert cpt % ib == 0 and ib % 8 == 0

    @functools.partial(
        pl.kernel,
        out_type=jax.ShapeDtypeStruct((NC, NPAD, 128), jnp.float32),
        mesh=_mesh,
        scratch_types=[
            pltpu.VMEM((ib, CH), jnp.int32),
            pltpu.VMEM((ib, CH), jnp.int32),
            pltpu.VMEM((CH, width), jnp.float32),
            pltpu.VMEM((CH, width), jnp.float32),
            pltpu.VMEM_SHARED((NPAD, width), jnp.float32),
            pltpu.VMEM_SHARED((NPAD, width), jnp.float32),
            pltpu.SemaphoreType.DMA,
            pltpu.SemaphoreType.DMA,
            pltpu.SemaphoreType.DMA,
            pltpu.SemaphoreType.DMA,
        ],
        compiler_params=_cp_linear,
    )
    def seg(taba_hbm, tabb_hbm, src_hbm, dst_hbm, out_hbm, sidx, didx,
            rows0, rows1, tab, acc, gs0, gs1, ss0, ss1):
        c = lax.axis_index("c")
        s = lax.axis_index("s")
        off = s * cpt if split else (c * NS + s) * cpt
        bufs = ((rows0, gs0, ss0), (rows1, gs1, ss1))
        nb = len(bufs)

        @pl.loop(0, CH)
        def _(i):
            @pl.loop(0, width, step=L)
            def _(j):
                rows0[i, pl.ds(j, L)] = jnp.zeros((L,), jnp.float32)

        @pl.loop(0, RSUB, step=CH)
        def _(r):
            pltpu.sync_copy(rows0, acc.at[pl.ds(s * RSUB + r, CH)])

        rsl = pl.ds(s * RSUB, RSUB)
        if split:
            @pl.when(c == 0)
            def _():
                pltpu.sync_copy(taba_hbm.at[rsl, pl.ds(0, width)],
                                tab.at[rsl])

            @pl.when(c == 1)
            def _():
                pltpu.sync_copy(taba_hbm.at[rsl, pl.ds(width, width)],
                                tab.at[rsl])
        else:
            pltpu.sync_copy(taba_hbm.at[rsl, pl.ds(0, width)], tab.at[rsl])
        plsc.subcore_barrier()

        @pl.loop(0, cpt // ib)
        def _(g):
            base = off + g * ib
            pltpu.sync_copy(src_hbm.at[pl.ds(base, ib)], sidx)
            pltpu.sync_copy(dst_hbm.at[pl.ds(base, ib)], didx)

            for b, (rb, gs, _) in enumerate(bufs):
                pltpu.async_copy(tab.at[sidx.at[b]], rb, gs)

            @pl.loop(0, ib, step=nb)
            def _(j):
                for b, (rb, gs, ss) in enumerate(bufs):
                    i = j + b
                    pltpu.make_async_copy(tab.at[sidx.at[i]], rb, gs).wait()
                    pltpu.async_copy(rb, acc.at[didx.at[i]], ss, add=True)

                    @pl.when(i + nb < ib)
                    def _():
                        pltpu.make_async_copy(rb, acc.at[didx.at[i]],
                                              ss).wait()
                        pltpu.async_copy(tab.at[sidx.at[i + nb]], rb, gs)

            for b, (rb, _, ss) in enumerate(bufs):
                pltpu.make_async_copy(rb, acc.at[didx.at[ib - nb + b]],
                                      ss).wait()

        plsc.subcore_barrier()
        pltpu.sync_copy(acc.at[rsl], out_hbm.at[c].at[rsl, pl.ds(0, width)])

    return seg


_sc_seg_sum_h64 = _make_sc_seg_sum(DIN // 2, 16, True)
_sc_seg_sum_48 = _make_sc_seg_sum(CPAD, 16, False)


_RB = 2048
_GRID = NPAD // _RB
_RBF = 1000
_GRIDF = N // _RBF


def _norm(col):
    return lax.rsqrt(jnp.maximum(col, 1.0))


def _prep_body(deg_ref, x_ref, xn_ref, nrm_ref):
    n_out = _norm(deg_ref[0, 0, :, 0:1] + deg_ref[1, 0, :, 0:1])
    n_in = _norm(deg_ref[0, 1, :, 0:1] + deg_ref[1, 1, :, 0:1])
    nrm_ref[:, 0:1] = n_out
    nrm_ref[:, 1:2] = n_in
    xn_ref[...] = x_ref[...] * n_out


def _tc_prep(degp, x_pad):
    return pl.pallas_call(
        _prep_body,
        grid=(_GRID,),
        in_specs=[
            pl.BlockSpec((NC, 2, _RB, 128), lambda i: (0, 0, i, 0)),
            pl.BlockSpec((_RB, DIN), lambda i: (i, 0)),
        ],
        out_specs=[
            pl.BlockSpec((_RB, DIN), lambda i: (i, 0)),
            pl.BlockSpec((_RB, 2), lambda i: (i, 0)),
        ],
        out_shape=[
            jax.ShapeDtypeStruct((NPAD, DIN), jnp.float32),
            jax.ShapeDtypeStruct((NPAD, 2), jnp.float32),
        ],
    )(degp, x_pad)


def _mid_body(nrm_ref, p_ref, w1_ref, b1_ref, w2_ref, o_ref):
    n_in = nrm_ref[:, 1:2]
    n_out = nrm_ref[:, 0:1]
    m = jnp.concatenate([p_ref[0][:, : DIN // 2], p_ref[1][:, : DIN // 2]],
                        axis=1) * n_in
    h = jnp.dot(m, w1_ref[...], preferred_element_type=jnp.float32)
    h = jnp.maximum(h + b1_ref[...], 0.0)
    z = jnp.dot(h, w2_ref[...], preferred_element_type=jnp.float32)
    o_ref[...] = z * n_out


def _tc_mid(norms, p, w1, b1, w2p):
    return pl.pallas_call(
        _mid_body,
        grid=(_GRID,),
        in_specs=[
            pl.BlockSpec((_RB, 2), lambda i: (i, 0)),
            pl.BlockSpec((NC, _RB, 128), lambda i: (0, i, 0)),
            pl.BlockSpec((DIN, HID), lambda i: (0, 0)),
            pl.BlockSpec((1, HID), lambda i: (0, 0)),
            pl.BlockSpec((HID, DIN), lambda i: (0, 0)),
        ],
        out_specs=pl.BlockSpec((_RB, DIN), lambda i: (i, 0)),
        out_shape=jax.ShapeDtypeStruct((NPAD, DIN), jnp.float32),
    )(norms, p, w1, b1, w2p)


def _final_body(nrm_ref, q_ref, b2_ref, o_ref):
    n_in = nrm_ref[:, 1:2]
    o_ref[...] = ((q_ref[0][:, :NCLS] + q_ref[1][:, :NCLS]) * n_in
                  + b2_ref[...])


def _tc_final(norms, q, b2p):
    return pl.pallas_call(
        _final_body,
        grid=(_GRIDF,),
        in_specs=[
            pl.BlockSpec((_RBF, 2), lambda i: (i, 0)),
            pl.BlockSpec((NC, _RBF, 128), lambda i: (0, i, 0)),
            pl.BlockSpec((1, NCLS), lambda i: (0, 0)),
        ],
        out_specs=pl.BlockSpec((_RBF, NCLS), lambda i: (i, 0)),
        out_shape=jax.ShapeDtypeStruct((N, NCLS), jnp.float32),
    )(norms, q, b2p)


def kernel(x, edge_index, W1, b1, W2, b2):
    ei = edge_index.astype(jnp.int32)
    pad = jnp.full((2, EPAD - E), N, dtype=jnp.int32)
    ep = jnp.concatenate([ei, pad], axis=1)
    src_rows = ep[0].reshape(EPAD // CH, CH)
    dst_rows = ep[1].reshape(EPAD // CH, CH)

    degp = _sc_degrees(src_rows, dst_rows)
    x_pad = jnp.pad(x, ((0, NPAD - N), (0, 0)))
    xn, norms = _tc_prep(degp, x_pad)
    p = _sc_seg_sum_h64(xn, xn, src_rows, dst_rows)

    w2p = jnp.pad(W2, ((0, 0), (0, DIN - NCLS)))
    zn = _tc_mid(norms, p, W1, b1.reshape(1, HID), w2p)
    q = _sc_seg_sum_48(zn, zn, src_rows, dst_rows)

    return _tc_final(norms, q, b2.reshape(1, NCLS))
